# baseline (speedup 1.0000x reference)
import jax
import jax.numpy as jnp
from jax import lax
from jax.experimental import pallas as pl
from jax.experimental.pallas import tpu as pltpu
from jax.experimental.pallas import tpu_sc as plsc

_N = 10000
_NP = 10240
_T = 7
_F = 9
_H = 128
_G3 = 3 * _H
_E = 640000
_NC, _NS = 2, 16
_NW = _NC * _NS
_CH = 116
_RB0 = 211
_RB1 = 135
_NCH = _NS * (_RB0 + _RB1)
_EPAD = _NCH * _CH
_RPS = _NP // _NS
_TILE = 512
_GRID = _NP // _TILE


def _make_sc_agg(with_deg):
  mesh = plsc.VectorSubcoreMesh(core_axis_name="c", subcore_axis_name="s",
                                num_cores=_NC, num_subcores=_NS)
  out_type = [jax.ShapeDtypeStruct((_NC, _NP, _H), jnp.float32)]
  scratch = (
      [pltpu.VMEM_SHARED((_NP, _H), jnp.float32)]
      + [pltpu.VMEM((2, _CH), jnp.int32) for _ in range(6)]
      + [pltpu.VMEM((_CH, _H), jnp.float32) for _ in range(3)]
      + [pltpu.SemaphoreType.DMA for _ in range(3)]
      + [pltpu.SemaphoreType.DMA for _ in range(3)]
      + [pltpu.SemaphoreType.DMA for _ in range(6)]
  )
  if with_deg:
    out_type.append(jax.ShapeDtypeStruct((_NC, _NP), jnp.float32))
    scratch += [
        pltpu.VMEM_SHARED((_NP,), jnp.float32),
        pltpu.VMEM((_CH,), jnp.float32),
        pltpu.SemaphoreType.DMA,
    ]

  def body(h_hbm, eidx_hbm, z2_hbm, z1_hbm, ones_hbm, *rest):
    if with_deg:
      out_hbm, deg_hbm = rest[0], rest[1]
      rest = rest[2:]
    else:
      out_hbm = rest[0]
      rest = rest[1:]
    acc = rest[0]
    idxs = rest[1:7]
    rows = rest[7:10]
    gsem = rest[10:13]
    ssem = rest[13:16]
    isem = rest[16:22]
    if with_deg:
      dacc, ones_v, dsem = rest[22:25]
    c = lax.axis_index("c")
    s = lax.axis_index("s")
    pltpu.sync_copy(z2_hbm.at[pl.ds(s * _RPS, _RPS)],
                    acc.at[pl.ds(s * _RPS, _RPS)])
    if with_deg:
      pltpu.sync_copy(z1_hbm.at[pl.ds(s * _RPS, _RPS)],
                      dacc.at[pl.ds(s * _RPS, _RPS)])
      pltpu.sync_copy(ones_hbm, ones_v)
    plsc.subcore_barrier()

    n = jnp.where(c == 0, _RB0, _RB1)
    m6 = n // 6
    t = n - 6 * m6
    cb = jnp.where(c == 0, s * _RB0, _NS * _RB0 + s * _RB1)

    def wait_idx(q, g):
      pltpu.make_async_copy(eidx_hbm.at[g], idxs[q], isem[q]).wait()

    def start_gather(q, r):
      pltpu.async_copy(h_hbm.at[idxs[q].at[0]], rows[r], gsem[r])

    def wait_gather(q, r):
      pltpu.make_async_copy(h_hbm.at[idxs[q].at[0]], rows[r], gsem[r]).wait()

    def start_scatter(q, r):
      pltpu.async_copy(rows[r], acc.at[idxs[q].at[1]], ssem[r], add=True)
      if with_deg:
        pltpu.async_copy(ones_v, dacc.at[idxs[q].at[1]], dsem, add=True)

    def wait_scatter(q, r):
      pltpu.make_async_copy(rows[r], acc.at[idxs[q].at[1]], ssem[r]).wait()
      if with_deg:
        pltpu.make_async_copy(ones_v, dacc.at[idxs[q].at[1]], dsem).wait()

    pltpu.sync_copy(eidx_hbm.at[cb], idxs[0])
    pltpu.sync_copy(eidx_hbm.at[cb + 1], idxs[1])
    for q in (2, 3, 4):
      pltpu.async_copy(eidx_hbm.at[cb + q], idxs[q], isem[q])
    start_gather(0, 0)
    start_gather(1, 1)

    def iter6(i, carry):
      jl0 = 6 * i
      for b in range(6):
        r = b % 3
        jl = jl0 + b
        jg = cb + jl
        wait_gather(b, r)
        start_scatter(b, r)
        rp = (r + 2) % 3
        bp = (b + 5) % 6

        @pl.when(jl > 0)
        def _():
          wait_scatter(bp, rp)

        @pl.when(jl + 5 < n)
        def _():
          pltpu.async_copy(eidx_hbm.at[jg + 5], idxs[bp], isem[bp])

        bn = (b + 2) % 6

        @pl.when(jl + 2 < n)
        def _():
          wait_idx(bn, jg + 2)
          start_gather(bn, rp)

      return carry

    lax.fori_loop(0, m6, iter6, 0)

    for tb in range(5):
      r = tb % 3
      rp = (r + 2) % 3
      bp = (tb + 5) % 6
      bn = (tb + 2) % 6

      @pl.when(tb < t)
      def _():
        jl = 6 * m6 + tb
        jg = cb + jl
        wait_gather(tb, r)
        start_scatter(tb, r)
        wait_scatter(bp, rp)

        @pl.when(jl + 2 < n)
        def _():
          wait_idx(bn, jg + 2)
          start_gather(bn, rp)

    for tv in range(1, 6):
      @pl.when(t == tv)
      def _():
        wait_scatter((tv - 1) % 6, (tv - 1) % 3)

    plsc.subcore_barrier()
    pltpu.sync_copy(acc.at[pl.ds(s * _RPS, _RPS)],
                    out_hbm.at[c, pl.ds(s * _RPS, _RPS)])
    if with_deg:
      pltpu.sync_copy(dacc.at[pl.ds(s * _RPS, _RPS)],
                      deg_hbm.at[c, pl.ds(s * _RPS, _RPS)])

  return pl.kernel(body, out_type=out_type, mesh=mesh, scratch_types=scratch)


_SC_CACHE = {}


def _sc_agg_kernel(with_deg):
  if with_deg not in _SC_CACHE:
    _SC_CACHE[with_deg] = _make_sc_agg(with_deg)
  return _SC_CACHE[with_deg]


def _gru_body(x_ref, wih_ref, whh_ref, bih_ref, bhh_ref, h_ref):
  gi = jnp.dot(x_ref[...], wih_ref[...], preferred_element_type=jnp.float32)
  h = jnp.zeros((_TILE, _H), jnp.float32)
  for t in range(_T):
    gh = jnp.dot(h, whh_ref[...], preferred_element_type=jnp.float32) + bhh_ref[...]
    g = gi[:, t * _G3:(t + 1) * _G3] + bih_ref[...]
    r = jax.nn.sigmoid(g[:, :_H] + gh[:, :_H])
    z = jax.nn.sigmoid(g[:, _H:2 * _H] + gh[:, _H:2 * _H])
    n = jnp.tanh(g[:, 2 * _H:] + r * gh[:, 2 * _H:])
    h = (1.0 - z) * n + z * h
  h_ref[...] = h


def _gru(xf, wih_big, whhT, bih2, bhh2):
  return pl.pallas_call(
      _gru_body,
      grid=(_GRID,),
      in_specs=[
          pl.BlockSpec((_TILE, 64), lambda i: (i, 0)),
          pl.BlockSpec((64, _T * _G3), lambda i: (0, 0)),
          pl.BlockSpec((_H, _G3), lambda i: (0, 0)),
          pl.BlockSpec((1, _G3), lambda i: (0, 0)),
          pl.BlockSpec((1, _G3), lambda i: (0, 0)),
      ],
      out_specs=pl.BlockSpec((_TILE, _H), lambda i: (i, 0)),
      out_shape=jax.ShapeDtypeStruct((_NP, _H), jnp.float32),
  )(xf, wih_big, whhT, bih2, bhh2)


def _sage_core(p_ref, d_ref, h_ref, wl_ref, bl_ref, wr_ref, g_ref, b_ref):
  deg = jnp.maximum(d_ref[...][:, 0:1] + d_ref[...][:, 1:2], 1.0)
  agg = (p_ref[0] + p_ref[1]) / deg
  pre = (jnp.dot(agg, wl_ref[...], preferred_element_type=jnp.float32)
         + bl_ref[...]
         + jnp.dot(h_ref[...], wr_ref[...], preferred_element_type=jnp.float32))
  mu = jnp.mean(pre, axis=1, keepdims=True)
  var = jnp.mean((pre - mu) ** 2, axis=1, keepdims=True)
  y = (pre - mu) * lax.rsqrt(var + 1e-5) * g_ref[...] + b_ref[...]
  return jnp.maximum(y, 0.0)


def _sage_mid_body(p_ref, d_ref, h_ref, wl_ref, bl_ref, wr_ref, g_ref, b_ref,
                   o_ref):
  o_ref[...] = _sage_core(p_ref, d_ref, h_ref, wl_ref, bl_ref, wr_ref, g_ref,
                          b_ref)


def _sage_dec_body(p_ref, d_ref, h_ref, wl_ref, bl_ref, wr_ref, g_ref, b_ref,
                   dw_ref, db_ref, o_ref):
  y = _sage_core(p_ref, d_ref, h_ref, wl_ref, bl_ref, wr_ref, g_ref, b_ref)
  o_ref[...] = (jnp.dot(y, dw_ref[...], preferred_element_type=jnp.float32)
                + db_ref[...])


def _sage(body, extra, p, d2, h, wlT, bl2, wrT, g2, b2):
  n_extra = len(extra)
  in_specs = [
      pl.BlockSpec((_NC, _TILE, _H), lambda i: (0, i, 0)),
      pl.BlockSpec((_TILE, _NC), lambda i: (i, 0)),
      pl.BlockSpec((_TILE, _H), lambda i: (i, 0)),
      pl.BlockSpec((_H, _H), lambda i: (0, 0)),
      pl.BlockSpec((1, _H), lambda i: (0, 0)),
      pl.BlockSpec((_H, _H), lambda i: (0, 0)),
      pl.BlockSpec((1, _H), lambda i: (0, 0)),
      pl.BlockSpec((1, _H), lambda i: (0, 0)),
  ] + [pl.BlockSpec((_H, _H), lambda i: (0, 0)),
       pl.BlockSpec((1, _H), lambda i: (0, 0))][:n_extra]
  return pl.pallas_call(
      body,
      grid=(_GRID,),
      in_specs=in_specs,
      out_specs=pl.BlockSpec((_TILE, _H), lambda i: (i, 0)),
      out_shape=jax.ShapeDtypeStruct((_NP, _H), jnp.float32),
  )(p, d2, h, wlT, bl2, wrT, g2, b2, *extra)


def _impl(x_seq, edge_index, gru_W_ih, gru_W_hh, gru_b_ih, gru_b_hh,
          sage1_Wl, sage1_bl, sage1_Wr, sage2_Wl, sage2_bl, sage2_Wr,
          ln1_g, ln1_b, ln2_g, ln2_b, dec_W, dec_b):
  f32 = jnp.float32
  npad = _EPAD - _E
  src = jnp.concatenate(
      [edge_index[0].astype(jnp.int32), jnp.zeros((npad,), jnp.int32)])
  dst = jnp.concatenate(
      [edge_index[1].astype(jnp.int32), jnp.full((npad,), _N, jnp.int32)])
  eidx = jnp.stack([src.reshape(-1, _CH), dst.reshape(-1, _CH)], axis=1)

  xf = jnp.zeros((_NP, 64), f32).at[:_N, :_T * _F].set(
      x_seq.reshape(_N, _T * _F))
  wih_big = jnp.zeros((64, _T * _G3), f32)
  wt = gru_W_ih.T
  for t in range(_T):
    wih_big = wih_big.at[t * _F:(t + 1) * _F, t * _G3:(t + 1) * _G3].set(wt)
  whhT = gru_W_hh.T
  bih2 = gru_b_ih[None, :]
  bhh2 = gru_b_hh[None, :]

  z2 = jnp.zeros((_NP, _H), f32)
  z1 = jnp.zeros((_NP,), f32)
  ones = jnp.ones((_CH,), f32)

  h0 = _gru(xf, wih_big, whhT, bih2, bhh2)

  p1, dpart = _sc_agg_kernel(True)(h0, eidx, z2, z1, ones)
  d2 = dpart.T

  h1 = _sage(_sage_mid_body, (), p1, d2, h0,
             sage1_Wl.T, sage1_bl[None, :], sage1_Wr.T,
             ln1_g[None, :], ln1_b[None, :])

  (p2,) = _sc_agg_kernel(False)(h1, eidx, z2, z1, ones)

  dwT = jnp.zeros((_H, _H), f32).at[:, :3].set(dec_W.T)
  db2 = jnp.zeros((1, _H), f32).at[0, :3].set(dec_b)
  out = _sage(_sage_dec_body, (dwT, db2), p2, d2, h1,
              sage2_Wl.T, sage2_bl[None, :], sage2_Wr.T,
              ln2_g[None, :], ln2_b[None, :])
  return out[:_N, :3]


kernel = jax.jit(_impl)


# speedup vs baseline: 1.0145x; 1.0145x over previous
import jax
import jax.numpy as jnp
from jax import lax
from jax.experimental import pallas as pl
from jax.experimental.pallas import tpu as pltpu
from jax.experimental.pallas import tpu_sc as plsc

_N = 10000
_NP = 10240
_T = 7
_F = 9
_H = 128
_G3 = 3 * _H
_E = 640000
_NC, _NS = 2, 16
_NW = _NC * _NS
_CH = 116
_RB0 = 219
_RB1 = 127
_NCH = _NS * (_RB0 + _RB1)
_EPAD = _NCH * _CH
_RPS = _NP // _NS
_TILE = 512
_GRID = _NP // _TILE


def _make_sc_agg(with_deg):
  mesh = plsc.VectorSubcoreMesh(core_axis_name="c", subcore_axis_name="s",
                                num_cores=_NC, num_subcores=_NS)
  out_type = [jax.ShapeDtypeStruct((_NC, _NP, _H), jnp.float32)]
  scratch = (
      [pltpu.VMEM_SHARED((_NP, _H), jnp.float32)]
      + [pltpu.VMEM((2, _CH), jnp.int32) for _ in range(6)]
      + [pltpu.VMEM((_CH, _H), jnp.float32) for _ in range(3)]
      + [pltpu.SemaphoreType.DMA for _ in range(3)]
      + [pltpu.SemaphoreType.DMA for _ in range(3)]
      + [pltpu.SemaphoreType.DMA for _ in range(6)]
  )
  if with_deg:
    out_type.append(jax.ShapeDtypeStruct((_NC, _NP), jnp.float32))
    scratch += [
        pltpu.VMEM_SHARED((_NP,), jnp.float32),
        pltpu.VMEM((_CH,), jnp.float32),
        pltpu.SemaphoreType.DMA,
    ]

  def body(h_hbm, eidx_hbm, z2_hbm, z1_hbm, ones_hbm, *rest):
    if with_deg:
      out_hbm, deg_hbm = rest[0], rest[1]
      rest = rest[2:]
    else:
      out_hbm = rest[0]
      rest = rest[1:]
    acc = rest[0]
    idxs = rest[1:7]
    rows = rest[7:10]
    gsem = rest[10:13]
    ssem = rest[13:16]
    isem = rest[16:22]
    if with_deg:
      dacc, ones_v, dsem = rest[22:25]
    c = lax.axis_index("c")
    s = lax.axis_index("s")
    pltpu.sync_copy(z2_hbm.at[pl.ds(s * _RPS, _RPS)],
                    acc.at[pl.ds(s * _RPS, _RPS)])
    if with_deg:
      pltpu.sync_copy(z1_hbm.at[pl.ds(s * _RPS, _RPS)],
                      dacc.at[pl.ds(s * _RPS, _RPS)])
      pltpu.sync_copy(ones_hbm, ones_v)
    plsc.subcore_barrier()

    n = jnp.where(c == 0, _RB0, _RB1)
    m6 = n // 6
    t = n - 6 * m6
    cb = jnp.where(c == 0, s * _RB0, _NS * _RB0 + s * _RB1)

    def wait_idx(q, g):
      pltpu.make_async_copy(eidx_hbm.at[g], idxs[q], isem[q]).wait()

    def start_gather(q, r):
      pltpu.async_copy(h_hbm.at[idxs[q].at[0]], rows[r], gsem[r])

    def wait_gather(q, r):
      pltpu.make_async_copy(h_hbm.at[idxs[q].at[0]], rows[r], gsem[r]).wait()

    def start_scatter(q, r):
      pltpu.async_copy(rows[r], acc.at[idxs[q].at[1]], ssem[r], add=True)
      if with_deg:
        pltpu.async_copy(ones_v, dacc.at[idxs[q].at[1]], dsem, add=True)

    def wait_scatter(q, r):
      pltpu.make_async_copy(rows[r], acc.at[idxs[q].at[1]], ssem[r]).wait()
      if with_deg:
        pltpu.make_async_copy(ones_v, dacc.at[idxs[q].at[1]], dsem).wait()

    pltpu.sync_copy(eidx_hbm.at[cb], idxs[0])
    pltpu.sync_copy(eidx_hbm.at[cb + 1], idxs[1])
    for q in (2, 3, 4):
      pltpu.async_copy(eidx_hbm.at[cb + q], idxs[q], isem[q])
    start_gather(0, 0)
    start_gather(1, 1)

    def iter6(i, carry):
      jl0 = 6 * i
      for b in range(6):
        r = b % 3
        jl = jl0 + b
        jg = cb + jl
        wait_gather(b, r)
        start_scatter(b, r)
        rp = (r + 2) % 3
        bp = (b + 5) % 6

        @pl.when(jl > 0)
        def _():
          wait_scatter(bp, rp)

        @pl.when(jl + 5 < n)
        def _():
          pltpu.async_copy(eidx_hbm.at[jg + 5], idxs[bp], isem[bp])

        bn = (b + 2) % 6

        @pl.when(jl + 2 < n)
        def _():
          wait_idx(bn, jg + 2)
          start_gather(bn, rp)

      return carry

    lax.fori_loop(0, m6, iter6, 0)

    for tb in range(5):
      r = tb % 3
      rp = (r + 2) % 3
      bp = (tb + 5) % 6
      bn = (tb + 2) % 6

      @pl.when(tb < t)
      def _():
        jl = 6 * m6 + tb
        jg = cb + jl
        wait_gather(tb, r)
        start_scatter(tb, r)
        wait_scatter(bp, rp)

        @pl.when(jl + 2 < n)
        def _():
          wait_idx(bn, jg + 2)
          start_gather(bn, rp)

    for tv in range(1, 6):
      @pl.when(t == tv)
      def _():
        wait_scatter((tv - 1) % 6, (tv - 1) % 3)

    plsc.subcore_barrier()
    pltpu.sync_copy(acc.at[pl.ds(s * _RPS, _RPS)],
                    out_hbm.at[c, pl.ds(s * _RPS, _RPS)])
    if with_deg:
      pltpu.sync_copy(dacc.at[pl.ds(s * _RPS, _RPS)],
                      deg_hbm.at[c, pl.ds(s * _RPS, _RPS)])

  return pl.kernel(body, out_type=out_type, mesh=mesh, scratch_types=scratch)


_SC_CACHE = {}


def _sc_agg_kernel(with_deg):
  if with_deg not in _SC_CACHE:
    _SC_CACHE[with_deg] = _make_sc_agg(with_deg)
  return _SC_CACHE[with_deg]


def _gru_body(x_ref, wih_ref, whh_ref, bih_ref, bhh_ref, h_ref):
  gi = jnp.dot(x_ref[...], wih_ref[...], preferred_element_type=jnp.float32)
  h = jnp.zeros((_TILE, _H), jnp.float32)
  for t in range(_T):
    gh = jnp.dot(h, whh_ref[...], preferred_element_type=jnp.float32) + bhh_ref[...]
    g = gi[:, t * _G3:(t + 1) * _G3] + bih_ref[...]
    r = jax.nn.sigmoid(g[:, :_H] + gh[:, :_H])
    z = jax.nn.sigmoid(g[:, _H:2 * _H] + gh[:, _H:2 * _H])
    n = jnp.tanh(g[:, 2 * _H:] + r * gh[:, 2 * _H:])
    h = (1.0 - z) * n + z * h
  h_ref[...] = h


def _gru(xf, wih_big, whhT, bih2, bhh2):
  return pl.pallas_call(
      _gru_body,
      grid=(_GRID,),
      in_specs=[
          pl.BlockSpec((_TILE, 64), lambda i: (i, 0)),
          pl.BlockSpec((64, _T * _G3), lambda i: (0, 0)),
          pl.BlockSpec((_H, _G3), lambda i: (0, 0)),
          pl.BlockSpec((1, _G3), lambda i: (0, 0)),
          pl.BlockSpec((1, _G3), lambda i: (0, 0)),
      ],
      out_specs=pl.BlockSpec((_TILE, _H), lambda i: (i, 0)),
      out_shape=jax.ShapeDtypeStruct((_NP, _H), jnp.float32),
  )(xf, wih_big, whhT, bih2, bhh2)


def _sage_core(p_ref, d_ref, h_ref, wl_ref, bl_ref, wr_ref, g_ref, b_ref):
  deg = jnp.maximum(d_ref[...][:, 0:1] + d_ref[...][:, 1:2], 1.0)
  agg = (p_ref[0] + p_ref[1]) / deg
  pre = (jnp.dot(agg, wl_ref[...], preferred_element_type=jnp.float32)
         + bl_ref[...]
         + jnp.dot(h_ref[...], wr_ref[...], preferred_element_type=jnp.float32))
  mu = jnp.mean(pre, axis=1, keepdims=True)
  var = jnp.mean((pre - mu) ** 2, axis=1, keepdims=True)
  y = (pre - mu) * lax.rsqrt(var + 1e-5) * g_ref[...] + b_ref[...]
  return jnp.maximum(y, 0.0)


def _sage_mid_body(p_ref, d_ref, h_ref, wl_ref, bl_ref, wr_ref, g_ref, b_ref,
                   o_ref):
  o_ref[...] = _sage_core(p_ref, d_ref, h_ref, wl_ref, bl_ref, wr_ref, g_ref,
                          b_ref)


def _sage_dec_body(p_ref, d_ref, h_ref, wl_ref, bl_ref, wr_ref, g_ref, b_ref,
                   dw_ref, db_ref, o_ref):
  y = _sage_core(p_ref, d_ref, h_ref, wl_ref, bl_ref, wr_ref, g_ref, b_ref)
  o_ref[...] = (jnp.dot(y, dw_ref[...], preferred_element_type=jnp.float32)
                + db_ref[...])


def _sage(body, extra, p, d2, h, wlT, bl2, wrT, g2, b2):
  n_extra = len(extra)
  in_specs = [
      pl.BlockSpec((_NC, _TILE, _H), lambda i: (0, i, 0)),
      pl.BlockSpec((_TILE, _NC), lambda i: (i, 0)),
      pl.BlockSpec((_TILE, _H), lambda i: (i, 0)),
      pl.BlockSpec((_H, _H), lambda i: (0, 0)),
      pl.BlockSpec((1, _H), lambda i: (0, 0)),
      pl.BlockSpec((_H, _H), lambda i: (0, 0)),
      pl.BlockSpec((1, _H), lambda i: (0, 0)),
      pl.BlockSpec((1, _H), lambda i: (0, 0)),
  ] + [pl.BlockSpec((_H, _H), lambda i: (0, 0)),
       pl.BlockSpec((1, _H), lambda i: (0, 0))][:n_extra]
  return pl.pallas_call(
      body,
      grid=(_GRID,),
      in_specs=in_specs,
      out_specs=pl.BlockSpec((_TILE, _H), lambda i: (i, 0)),
      out_shape=jax.ShapeDtypeStruct((_NP, _H), jnp.float32),
  )(p, d2, h, wlT, bl2, wrT, g2, b2, *extra)


def _impl(x_seq, edge_index, gru_W_ih, gru_W_hh, gru_b_ih, gru_b_hh,
          sage1_Wl, sage1_bl, sage1_Wr, sage2_Wl, sage2_bl, sage2_Wr,
          ln1_g, ln1_b, ln2_g, ln2_b, dec_W, dec_b):
  f32 = jnp.float32
  npad = _EPAD - _E
  src = jnp.concatenate(
      [edge_index[0].astype(jnp.int32), jnp.zeros((npad,), jnp.int32)])
  dst = jnp.concatenate(
      [edge_index[1].astype(jnp.int32), jnp.full((npad,), _N, jnp.int32)])
  eidx = jnp.stack([src.reshape(-1, _CH), dst.reshape(-1, _CH)], axis=1)

  xf = jnp.zeros((_NP, 64), f32).at[:_N, :_T * _F].set(
      x_seq.reshape(_N, _T * _F))
  wih_big = jnp.zeros((64, _T * _G3), f32)
  wt = gru_W_ih.T
  for t in range(_T):
    wih_big = wih_big.at[t * _F:(t + 1) * _F, t * _G3:(t + 1) * _G3].set(wt)
  whhT = gru_W_hh.T
  bih2 = gru_b_ih[None, :]
  bhh2 = gru_b_hh[None, :]

  z2 = jnp.zeros((_NP, _H), f32)
  z1 = jnp.zeros((_NP,), f32)
  ones = jnp.ones((_CH,), f32)

  h0 = _gru(xf, wih_big, whhT, bih2, bhh2)

  p1, dpart = _sc_agg_kernel(True)(h0, eidx, z2, z1, ones)
  d2 = dpart.T

  h1 = _sage(_sage_mid_body, (), p1, d2, h0,
             sage1_Wl.T, sage1_bl[None, :], sage1_Wr.T,
             ln1_g[None, :], ln1_b[None, :])

  (p2,) = _sc_agg_kernel(False)(h1, eidx, z2, z1, ones)

  dwT = jnp.zeros((_H, _H), f32).at[:, :3].set(dec_W.T)
  db2 = jnp.zeros((1, _H), f32).at[0, :3].set(dec_b)
  out = _sage(_sage_dec_body, (dwT, db2), p2, d2, h1,
              sage2_Wl.T, sage2_bl[None, :], sage2_Wr.T,
              ln2_g[None, :], ln2_b[None, :])
  return out[:_N, :3]


kernel = jax.jit(_impl)


# speedup vs baseline: 1.0361x; 1.0213x over previous
import jax
import jax.numpy as jnp
from jax import lax
from jax.experimental import pallas as pl
from jax.experimental.pallas import tpu as pltpu
from jax.experimental.pallas import tpu_sc as plsc

_N = 10000
_NP = 10240
_T = 7
_F = 9
_H = 128
_G3 = 3 * _H
_E = 640000
_NC, _NS = 2, 16
_NW = _NC * _NS
_CH = 116
_RB0 = 227
_RB1 = 119
_NCH = _NS * (_RB0 + _RB1)
_EPAD = _NCH * _CH
_RPS = _NP // _NS
_TILE = 512
_GRID = _NP // _TILE


def _make_sc_agg(with_deg):
  mesh = plsc.VectorSubcoreMesh(core_axis_name="c", subcore_axis_name="s",
                                num_cores=_NC, num_subcores=_NS)
  out_type = [jax.ShapeDtypeStruct((_NC, _NP, _H), jnp.float32)]
  scratch = (
      [pltpu.VMEM_SHARED((_NP, _H), jnp.float32)]
      + [pltpu.VMEM((2, _CH), jnp.int32) for _ in range(6)]
      + [pltpu.VMEM((_CH, _H), jnp.float32) for _ in range(3)]
      + [pltpu.SemaphoreType.DMA for _ in range(3)]
      + [pltpu.SemaphoreType.DMA for _ in range(3)]
      + [pltpu.SemaphoreType.DMA for _ in range(6)]
  )
  if with_deg:
    out_type.append(jax.ShapeDtypeStruct((_NC, _NP), jnp.float32))
    scratch += [
        pltpu.VMEM_SHARED((_NP,), jnp.float32),
        pltpu.VMEM((_CH,), jnp.float32),
        pltpu.SemaphoreType.DMA,
    ]

  def body(h_hbm, eidx_hbm, z2_hbm, z1_hbm, ones_hbm, *rest):
    if with_deg:
      out_hbm, deg_hbm = rest[0], rest[1]
      rest = rest[2:]
    else:
      out_hbm = rest[0]
      rest = rest[1:]
    acc = rest[0]
    idxs = rest[1:7]
    rows = rest[7:10]
    gsem = rest[10:13]
    ssem = rest[13:16]
    isem = rest[16:22]
    if with_deg:
      dacc, ones_v, dsem = rest[22:25]
    c = lax.axis_index("c")
    s = lax.axis_index("s")
    pltpu.sync_copy(z2_hbm.at[pl.ds(s * _RPS, _RPS)],
                    acc.at[pl.ds(s * _RPS, _RPS)])
    if with_deg:
      pltpu.sync_copy(z1_hbm.at[pl.ds(s * _RPS, _RPS)],
                      dacc.at[pl.ds(s * _RPS, _RPS)])
      pltpu.sync_copy(ones_hbm, ones_v)
    plsc.subcore_barrier()

    n = jnp.where(c == 0, _RB0, _RB1)
    m6 = n // 6
    t = n - 6 * m6
    cb = jnp.where(c == 0, s * _RB0, _NS * _RB0 + s * _RB1)

    def wait_idx(q, g):
      pltpu.make_async_copy(eidx_hbm.at[g], idxs[q], isem[q]).wait()

    def start_gather(q, r):
      pltpu.async_copy(h_hbm.at[idxs[q].at[0]], rows[r], gsem[r])

    def wait_gather(q, r):
      pltpu.make_async_copy(h_hbm.at[idxs[q].at[0]], rows[r], gsem[r]).wait()

    def start_scatter(q, r):
      pltpu.async_copy(rows[r], acc.at[idxs[q].at[1]], ssem[r], add=True)
      if with_deg:
        pltpu.async_copy(ones_v, dacc.at[idxs[q].at[1]], dsem, add=True)

    def wait_scatter(q, r):
      pltpu.make_async_copy(rows[r], acc.at[idxs[q].at[1]], ssem[r]).wait()
      if with_deg:
        pltpu.make_async_copy(ones_v, dacc.at[idxs[q].at[1]], dsem).wait()

    pltpu.sync_copy(eidx_hbm.at[cb], idxs[0])
    pltpu.sync_copy(eidx_hbm.at[cb + 1], idxs[1])
    for q in (2, 3, 4):
      pltpu.async_copy(eidx_hbm.at[cb + q], idxs[q], isem[q])
    start_gather(0, 0)
    start_gather(1, 1)

    def iter6(i, carry):
      jl0 = 6 * i
      for b in range(6):
        r = b % 3
        jl = jl0 + b
        jg = cb + jl
        wait_gather(b, r)
        start_scatter(b, r)
        rp = (r + 2) % 3
        bp = (b + 5) % 6

        @pl.when(jl > 0)
        def _():
          wait_scatter(bp, rp)

        @pl.when(jl + 5 < n)
        def _():
          pltpu.async_copy(eidx_hbm.at[jg + 5], idxs[bp], isem[bp])

        bn = (b + 2) % 6

        @pl.when(jl + 2 < n)
        def _():
          wait_idx(bn, jg + 2)
          start_gather(bn, rp)

      return carry

    lax.fori_loop(0, m6, iter6, 0)

    for tb in range(5):
      r = tb % 3
      rp = (r + 2) % 3
      bp = (tb + 5) % 6
      bn = (tb + 2) % 6

      @pl.when(tb < t)
      def _():
        jl = 6 * m6 + tb
        jg = cb + jl
        wait_gather(tb, r)
        start_scatter(tb, r)
        wait_scatter(bp, rp)

        @pl.when(jl + 2 < n)
        def _():
          wait_idx(bn, jg + 2)
          start_gather(bn, rp)

    for tv in range(1, 6):
      @pl.when(t == tv)
      def _():
        wait_scatter((tv - 1) % 6, (tv - 1) % 3)

    plsc.subcore_barrier()
    pltpu.sync_copy(acc.at[pl.ds(s * _RPS, _RPS)],
                    out_hbm.at[c, pl.ds(s * _RPS, _RPS)])
    if with_deg:
      pltpu.sync_copy(dacc.at[pl.ds(s * _RPS, _RPS)],
                      deg_hbm.at[c, pl.ds(s * _RPS, _RPS)])

  return pl.kernel(body, out_type=out_type, mesh=mesh, scratch_types=scratch)


_SC_CACHE = {}


def _sc_agg_kernel(with_deg):
  if with_deg not in _SC_CACHE:
    _SC_CACHE[with_deg] = _make_sc_agg(with_deg)
  return _SC_CACHE[with_deg]


def _gru_body(x_ref, wih_ref, whh_ref, bih_ref, bhh_ref, h_ref):
  gi = jnp.dot(x_ref[...], wih_ref[...], preferred_element_type=jnp.float32)
  h = jnp.zeros((_TILE, _H), jnp.float32)
  for t in range(_T):
    gh = jnp.dot(h, whh_ref[...], preferred_element_type=jnp.float32) + bhh_ref[...]
    g = gi[:, t * _G3:(t + 1) * _G3] + bih_ref[...]
    r = jax.nn.sigmoid(g[:, :_H] + gh[:, :_H])
    z = jax.nn.sigmoid(g[:, _H:2 * _H] + gh[:, _H:2 * _H])
    n = jnp.tanh(g[:, 2 * _H:] + r * gh[:, 2 * _H:])
    h = (1.0 - z) * n + z * h
  h_ref[...] = h


def _gru(xf, wih_big, whhT, bih2, bhh2):
  return pl.pallas_call(
      _gru_body,
      grid=(_GRID,),
      in_specs=[
          pl.BlockSpec((_TILE, 64), lambda i: (i, 0)),
          pl.BlockSpec((64, _T * _G3), lambda i: (0, 0)),
          pl.BlockSpec((_H, _G3), lambda i: (0, 0)),
          pl.BlockSpec((1, _G3), lambda i: (0, 0)),
          pl.BlockSpec((1, _G3), lambda i: (0, 0)),
      ],
      out_specs=pl.BlockSpec((_TILE, _H), lambda i: (i, 0)),
      out_shape=jax.ShapeDtypeStruct((_NP, _H), jnp.float32),
  )(xf, wih_big, whhT, bih2, bhh2)


def _sage_core(p_ref, d_ref, h_ref, wl_ref, bl_ref, wr_ref, g_ref, b_ref):
  deg = jnp.maximum(d_ref[...][:, 0:1] + d_ref[...][:, 1:2], 1.0)
  agg = (p_ref[0] + p_ref[1]) / deg
  pre = (jnp.dot(agg, wl_ref[...], preferred_element_type=jnp.float32)
         + bl_ref[...]
         + jnp.dot(h_ref[...], wr_ref[...], preferred_element_type=jnp.float32))
  mu = jnp.mean(pre, axis=1, keepdims=True)
  var = jnp.mean((pre - mu) ** 2, axis=1, keepdims=True)
  y = (pre - mu) * lax.rsqrt(var + 1e-5) * g_ref[...] + b_ref[...]
  return jnp.maximum(y, 0.0)


def _sage_mid_body(p_ref, d_ref, h_ref, wl_ref, bl_ref, wr_ref, g_ref, b_ref,
                   o_ref):
  o_ref[...] = _sage_core(p_ref, d_ref, h_ref, wl_ref, bl_ref, wr_ref, g_ref,
                          b_ref)


def _sage_dec_body(p_ref, d_ref, h_ref, wl_ref, bl_ref, wr_ref, g_ref, b_ref,
                   dw_ref, db_ref, o_ref):
  y = _sage_core(p_ref, d_ref, h_ref, wl_ref, bl_ref, wr_ref, g_ref, b_ref)
  o_ref[...] = (jnp.dot(y, dw_ref[...], preferred_element_type=jnp.float32)
                + db_ref[...])


def _sage(body, extra, p, d2, h, wlT, bl2, wrT, g2, b2):
  n_extra = len(extra)
  in_specs = [
      pl.BlockSpec((_NC, _TILE, _H), lambda i: (0, i, 0)),
      pl.BlockSpec((_TILE, _NC), lambda i: (i, 0)),
      pl.BlockSpec((_TILE, _H), lambda i: (i, 0)),
      pl.BlockSpec((_H, _H), lambda i: (0, 0)),
      pl.BlockSpec((1, _H), lambda i: (0, 0)),
      pl.BlockSpec((_H, _H), lambda i: (0, 0)),
      pl.BlockSpec((1, _H), lambda i: (0, 0)),
      pl.BlockSpec((1, _H), lambda i: (0, 0)),
  ] + [pl.BlockSpec((_H, _H), lambda i: (0, 0)),
       pl.BlockSpec((1, _H), lambda i: (0, 0))][:n_extra]
  return pl.pallas_call(
      body,
      grid=(_GRID,),
      in_specs=in_specs,
      out_specs=pl.BlockSpec((_TILE, _H), lambda i: (i, 0)),
      out_shape=jax.ShapeDtypeStruct((_NP, _H), jnp.float32),
  )(p, d2, h, wlT, bl2, wrT, g2, b2, *extra)


def _impl(x_seq, edge_index, gru_W_ih, gru_W_hh, gru_b_ih, gru_b_hh,
          sage1_Wl, sage1_bl, sage1_Wr, sage2_Wl, sage2_bl, sage2_Wr,
          ln1_g, ln1_b, ln2_g, ln2_b, dec_W, dec_b):
  f32 = jnp.float32
  npad = _EPAD - _E
  src = jnp.concatenate(
      [edge_index[0].astype(jnp.int32), jnp.zeros((npad,), jnp.int32)])
  dst = jnp.concatenate(
      [edge_index[1].astype(jnp.int32), jnp.full((npad,), _N, jnp.int32)])
  eidx = jnp.stack([src.reshape(-1, _CH), dst.reshape(-1, _CH)], axis=1)

  xf = jnp.zeros((_NP, 64), f32).at[:_N, :_T * _F].set(
      x_seq.reshape(_N, _T * _F))
  wih_big = jnp.zeros((64, _T * _G3), f32)
  wt = gru_W_ih.T
  for t in range(_T):
    wih_big = wih_big.at[t * _F:(t + 1) * _F, t * _G3:(t + 1) * _G3].set(wt)
  whhT = gru_W_hh.T
  bih2 = gru_b_ih[None, :]
  bhh2 = gru_b_hh[None, :]

  z2 = jnp.zeros((_NP, _H), f32)
  z1 = jnp.zeros((_NP,), f32)
  ones = jnp.ones((_CH,), f32)

  h0 = _gru(xf, wih_big, whhT, bih2, bhh2)

  p1, dpart = _sc_agg_kernel(True)(h0, eidx, z2, z1, ones)
  d2 = dpart.T

  h1 = _sage(_sage_mid_body, (), p1, d2, h0,
             sage1_Wl.T, sage1_bl[None, :], sage1_Wr.T,
             ln1_g[None, :], ln1_b[None, :])

  (p2,) = _sc_agg_kernel(False)(h1, eidx, z2, z1, ones)

  dwT = jnp.zeros((_H, _H), f32).at[:, :3].set(dec_W.T)
  db2 = jnp.zeros((1, _H), f32).at[0, :3].set(dec_b)
  out = _sage(_sage_dec_body, (dwT, db2), p2, d2, h1,
              sage2_Wl.T, sage2_bl[None, :], sage2_Wr.T,
              ln2_g[None, :], ln2_b[None, :])
  return out[:_N, :3]


kernel = jax.jit(_impl)
